# Initial kernel scaffold; baseline (speedup 1.0000x reference)
import jax
import jax.numpy as jnp
from jax.experimental import pallas as pl


def kernel(tokens, E0, E1, E2, E3, Wp, bp):
    raise NotImplementedError("write your pallas kernel here")


# speedup vs baseline: 5.9595x; 5.9595x over previous
import functools

import jax
import jax.numpy as jnp
from jax import lax
from jax.experimental import pallas as pl
from jax.experimental.pallas import tpu as pltpu
from jax.experimental.pallas import tpu_sc as plsc

CARD = 100000
B, S, NF = 1024, 200, 4
BS = B * S
DIMS = (32, 32, 16, 16)
TOTAL_DIM = 96
EMBED_DIM = 128

NW = 32
LANES = 128
CHUNK_ROWS = 4
CHUNK = CHUNK_ROWS * LANES
NCHUNKS = BS // CHUNK
STEPS = -(-NCHUNKS // NW)


def _tc_precompute(E0, E1, E2, E3, Wp, bp2):
    R = 1000

    def mm(e0_ref, e1_ref, e2_ref, e3_ref, wp_ref, bp_ref,
           p0_ref, p1_ref, p2_ref, p3_ref):
        p0_ref[...] = jnp.dot(e0_ref[...], wp_ref[0:32, :],
                              preferred_element_type=jnp.float32) + bp_ref[...]
        p1_ref[...] = jnp.dot(e1_ref[...], wp_ref[32:64, :],
                              preferred_element_type=jnp.float32)
        p2_ref[...] = jnp.dot(e2_ref[...], wp_ref[64:80, :],
                              preferred_element_type=jnp.float32)
        p3_ref[...] = jnp.dot(e3_ref[...], wp_ref[80:96, :],
                              preferred_element_type=jnp.float32)

    pshape = jax.ShapeDtypeStruct((CARD, EMBED_DIM), jnp.float32)
    return pl.pallas_call(
        mm,
        grid=(CARD // R,),
        in_specs=[
            pl.BlockSpec((R, DIMS[0]), lambda i: (i, 0)),
            pl.BlockSpec((R, DIMS[1]), lambda i: (i, 0)),
            pl.BlockSpec((R, DIMS[2]), lambda i: (i, 0)),
            pl.BlockSpec((R, DIMS[3]), lambda i: (i, 0)),
            pl.BlockSpec((TOTAL_DIM, EMBED_DIM), lambda i: (0, 0)),
            pl.BlockSpec((1, EMBED_DIM), lambda i: (0, 0)),
        ],
        out_specs=[pl.BlockSpec((R, EMBED_DIM), lambda i: (i, 0))] * NF,
        out_shape=[pshape] * NF,
    )(E0, E1, E2, E3, Wp, bp2)


def _sc_gather_sum(idx, P0, P1, P2, P3):
    mesh = plsc.VectorSubcoreMesh(core_axis_name="c", subcore_axis_name="s")

    @functools.partial(
        pl.kernel,
        mesh=mesh,
        out_type=jax.ShapeDtypeStruct((BS, EMBED_DIM), jnp.float32),
        scratch_types=[
            pltpu.VMEM((NF, CHUNK_ROWS, LANES), jnp.int32),
            pltpu.VMEM((CHUNK, EMBED_DIM), jnp.float32),
            pltpu.SemaphoreType.DMA,
        ],
    )
    def gather_kernel(idx_hbm, t0, t1, t2, t3, out, idx_v, acc, sem):
        wid = lax.axis_index("c") * 16 + lax.axis_index("s")
        tabs = (t0, t1, t2, t3)

        def body(k, carry):
            cid = wid + k * NW

            @pl.when(cid < NCHUNKS)
            def _():
                pltpu.sync_copy(idx_hbm.at[:, cid], idx_v)
                first = []
                for j in range(CHUNK_ROWS):
                    first.append(pltpu.async_copy(
                        tabs[0].at[idx_v.at[0, j]],
                        acc.at[pl.ds(j * LANES, LANES)],
                        sem))
                for cp in first:
                    cp.wait()
                rest = []
                for f in range(1, NF):
                    for j in range(CHUNK_ROWS):
                        rest.append(pltpu.async_copy(
                            tabs[f].at[idx_v.at[f, j]],
                            acc.at[pl.ds(j * LANES, LANES)],
                            sem, add=True))
                for cp in rest:
                    cp.wait()
                pltpu.sync_copy(acc, out.at[pl.ds(cid * CHUNK, CHUNK)])

            return carry

        lax.fori_loop(0, STEPS, body, 0)

    return gather_kernel(idx, P0, P1, P2, P3)


def kernel(tokens, E0, E1, E2, E3, Wp, bp):
    idx = tokens.astype(jnp.int32).reshape(BS, NF).T.reshape(
        NF, NCHUNKS, CHUNK_ROWS, LANES)
    P0, P1, P2, P3 = _tc_precompute(E0, E1, E2, E3, Wp,
                                    bp.reshape(1, EMBED_DIM))
    out = _sc_gather_sum(idx, P0, P1, P2, P3)
    return out.reshape(B, S, EMBED_DIM)


# speedup vs baseline: 6.1650x; 1.0345x over previous
import functools

import jax
import jax.numpy as jnp
from jax import lax
from jax.experimental import pallas as pl
from jax.experimental.pallas import tpu as pltpu
from jax.experimental.pallas import tpu_sc as plsc

CARD = 100000
B, S, NF = 1024, 200, 4
BS = B * S
DIMS = (32, 32, 16, 16)
TOTAL_DIM = 96
EMBED_DIM = 128

NW = 32
LANES = 128
CHUNK_ROWS = 2
CHUNK = CHUNK_ROWS * LANES
NCHUNKS = BS // CHUNK
CPW = NCHUNKS // NW
PAIRS = (CPW - 1) // 2


def _tc_precompute(E0, E1, E2, E3, Wp, bp2):
    R = 1000

    def mm(e0_ref, e1_ref, e2_ref, e3_ref, wp_ref, bp_ref,
           p0_ref, p1_ref, p2_ref, p3_ref):
        p0_ref[...] = jnp.dot(e0_ref[...], wp_ref[0:32, :],
                              preferred_element_type=jnp.float32) + bp_ref[...]
        p1_ref[...] = jnp.dot(e1_ref[...], wp_ref[32:64, :],
                              preferred_element_type=jnp.float32)
        p2_ref[...] = jnp.dot(e2_ref[...], wp_ref[64:80, :],
                              preferred_element_type=jnp.float32)
        p3_ref[...] = jnp.dot(e3_ref[...], wp_ref[80:96, :],
                              preferred_element_type=jnp.float32)

    pshape = jax.ShapeDtypeStruct((CARD, EMBED_DIM), jnp.float32)
    return pl.pallas_call(
        mm,
        grid=(CARD // R,),
        in_specs=[
            pl.BlockSpec((R, DIMS[0]), lambda i: (i, 0)),
            pl.BlockSpec((R, DIMS[1]), lambda i: (i, 0)),
            pl.BlockSpec((R, DIMS[2]), lambda i: (i, 0)),
            pl.BlockSpec((R, DIMS[3]), lambda i: (i, 0)),
            pl.BlockSpec((TOTAL_DIM, EMBED_DIM), lambda i: (0, 0)),
            pl.BlockSpec((1, EMBED_DIM), lambda i: (0, 0)),
        ],
        out_specs=[pl.BlockSpec((R, EMBED_DIM), lambda i: (i, 0))] * NF,
        out_shape=[pshape] * NF,
    )(E0, E1, E2, E3, Wp, bp2)


def _sc_gather_sum(idx, P0, P1, P2, P3):
    mesh = plsc.VectorSubcoreMesh(core_axis_name="c", subcore_axis_name="s")

    @functools.partial(
        pl.kernel,
        mesh=mesh,
        out_type=jax.ShapeDtypeStruct((BS, EMBED_DIM), jnp.float32),
        scratch_types=[
            pltpu.VMEM((NF, CPW, CHUNK_ROWS, LANES), jnp.int32),
            pltpu.VMEM((CHUNK, EMBED_DIM), jnp.float32),
            pltpu.VMEM((CHUNK, EMBED_DIM), jnp.float32),
            pltpu.SemaphoreType.DMA((6,)),
        ],
    )
    def gather_kernel(idx_hbm, t0, t1, t2, t3, out, idx_all, buf_a, buf_b,
                      sems):
        wid = lax.axis_index("c") * 16 + lax.axis_index("s")
        chunk0 = wid * CPW
        tabs = (t0, t1, t2, t3)
        bufs = (buf_a, buf_b)
        F0, ADD, OUT = 0, 2, 4

        for f in range(NF):
            pltpu.sync_copy(idx_hbm.at[f, pl.ds(chunk0, CPW)], idx_all.at[f])

        def fire_f0(b, k):
            for j in range(CHUNK_ROWS):
                pltpu.async_copy(tabs[0].at[idx_all.at[0, k, j]],
                                 bufs[b].at[pl.ds(j * LANES, LANES)],
                                 sems.at[F0 + b])

        def wait_f0(b):
            for j in range(CHUNK_ROWS):
                pltpu.make_async_copy(
                    tabs[0].at[idx_all.at[0, 0, j]],
                    bufs[b].at[pl.ds(j * LANES, LANES)],
                    sems.at[F0 + b]).wait()

        def fire_adds(b, k):
            for f in range(1, NF):
                for j in range(CHUNK_ROWS):
                    pltpu.async_copy(tabs[f].at[idx_all.at[f, k, j]],
                                     bufs[b].at[pl.ds(j * LANES, LANES)],
                                     sems.at[ADD + b], add=True)

        def wait_adds(b):
            for f in range(1, NF):
                for j in range(CHUNK_ROWS):
                    pltpu.make_async_copy(
                        tabs[f].at[idx_all.at[f, 0, j]],
                        bufs[b].at[pl.ds(j * LANES, LANES)],
                        sems.at[ADD + b]).wait()

        def fire_out(b, k):
            pltpu.async_copy(bufs[b],
                             out.at[pl.ds((chunk0 + k) * CHUNK, CHUNK)],
                             sems.at[OUT + b])

        def wait_out(b):
            pltpu.make_async_copy(bufs[b], out.at[pl.ds(0, CHUNK)],
                                  sems.at[OUT + b]).wait()

        fire_f0(0, 0)

        def body(g, carry):
            c0 = 2 * g
            c1 = c0 + 1
            wait_f0(0)
            fire_adds(0, c0)

            @pl.when(g > 0)
            def _():
                wait_out(1)

            fire_f0(1, c1)
            wait_adds(0)
            fire_out(0, c0)
            wait_f0(1)
            fire_adds(1, c1)
            wait_out(0)
            fire_f0(0, c0 + 2)
            wait_adds(1)
            fire_out(1, c1)
            return carry

        lax.fori_loop(0, PAIRS, body, 0)

        wait_f0(0)
        fire_adds(0, CPW - 1)
        wait_out(1)
        wait_adds(0)
        fire_out(0, CPW - 1)
        wait_out(0)

    return gather_kernel(idx, P0, P1, P2, P3)


def kernel(tokens, E0, E1, E2, E3, Wp, bp):
    idx = tokens.astype(jnp.int32).reshape(BS, NF).T.reshape(
        NF, NCHUNKS, CHUNK_ROWS, LANES)
    P0, P1, P2, P3 = _tc_precompute(E0, E1, E2, E3, Wp,
                                    bp.reshape(1, EMBED_DIM))
    out = _sc_gather_sum(idx, P0, P1, P2, P3)
    return out.reshape(B, S, EMBED_DIM)


# speedup vs baseline: 8.8237x; 1.4313x over previous
import functools

import jax
import jax.numpy as jnp
from jax import lax
from jax.experimental import pallas as pl
from jax.experimental.pallas import tpu as pltpu
from jax.experimental.pallas import tpu_sc as plsc

CARD = 100000
B, S, NF = 1024, 200, 4
BS = B * S
DIMS = (32, 32, 16, 16)
TOTAL_DIM = 96
EMBED_DIM = 128

NW = 32
LANES = 128
CHUNK_ROWS = 2
CHUNK = CHUNK_ROWS * LANES
NCHUNKS = BS // CHUNK
CPW = NCHUNKS // NW
PAIRS = (CPW - 1) // 2


def _tc_precompute(E0t, E1t, E2t, E3t, Wp, bp2):
    R = 1024
    cdims = (((0,), (0,)), ((), ()))

    def mm(e0_ref, e1_ref, e2_ref, e3_ref, wp_ref, bp_ref,
           p0_ref, p1_ref, p2_ref, p3_ref):
        p0_ref[...] = lax.dot_general(
            e0_ref[...], wp_ref[0:32, :], cdims,
            preferred_element_type=jnp.float32) + bp_ref[...]
        p1_ref[...] = lax.dot_general(
            e1_ref[...], wp_ref[32:64, :], cdims,
            preferred_element_type=jnp.float32)
        p2_ref[...] = lax.dot_general(
            e2_ref[...], wp_ref[64:80, :], cdims,
            preferred_element_type=jnp.float32)
        p3_ref[...] = lax.dot_general(
            e3_ref[...], wp_ref[80:96, :], cdims,
            preferred_element_type=jnp.float32)

    pshape = jax.ShapeDtypeStruct((CARD, EMBED_DIM), jnp.float32)
    return pl.pallas_call(
        mm,
        grid=(pl.cdiv(CARD, R),),
        in_specs=[
            pl.BlockSpec((DIMS[0], R), lambda i: (0, i)),
            pl.BlockSpec((DIMS[1], R), lambda i: (0, i)),
            pl.BlockSpec((DIMS[2], R), lambda i: (0, i)),
            pl.BlockSpec((DIMS[3], R), lambda i: (0, i)),
            pl.BlockSpec((TOTAL_DIM, EMBED_DIM), lambda i: (0, 0)),
            pl.BlockSpec((1, EMBED_DIM), lambda i: (0, 0)),
        ],
        out_specs=[pl.BlockSpec((R, EMBED_DIM), lambda i: (i, 0))] * NF,
        out_shape=[pshape] * NF,
    )(E0t, E1t, E2t, E3t, Wp, bp2)


def _sc_gather_sum(idx, P0, P1, P2, P3):
    mesh = plsc.VectorSubcoreMesh(core_axis_name="c", subcore_axis_name="s")

    @functools.partial(
        pl.kernel,
        mesh=mesh,
        out_type=jax.ShapeDtypeStruct((BS, EMBED_DIM), jnp.float32),
        scratch_types=[
            pltpu.VMEM((NF, CPW, CHUNK_ROWS, LANES), jnp.int32),
            pltpu.VMEM((CHUNK, EMBED_DIM), jnp.float32),
            pltpu.VMEM((CHUNK, EMBED_DIM), jnp.float32),
            pltpu.SemaphoreType.DMA((6,)),
        ],
    )
    def gather_kernel(idx_hbm, t0, t1, t2, t3, out, idx_all, buf_a, buf_b,
                      sems):
        wid = lax.axis_index("c") * 16 + lax.axis_index("s")
        chunk0 = wid * CPW
        tabs = (t0, t1, t2, t3)
        bufs = (buf_a, buf_b)
        F0, ADD, OUT = 0, 2, 4

        for f in range(NF):
            pltpu.sync_copy(idx_hbm.at[f, pl.ds(chunk0, CPW)], idx_all.at[f])

        def fire_f0(b, k):
            for j in range(CHUNK_ROWS):
                pltpu.async_copy(tabs[0].at[idx_all.at[0, k, j]],
                                 bufs[b].at[pl.ds(j * LANES, LANES)],
                                 sems.at[F0 + b])

        def wait_f0(b):
            for j in range(CHUNK_ROWS):
                pltpu.make_async_copy(
                    tabs[0].at[idx_all.at[0, 0, j]],
                    bufs[b].at[pl.ds(j * LANES, LANES)],
                    sems.at[F0 + b]).wait()

        def fire_adds(b, k):
            for f in range(1, NF):
                for j in range(CHUNK_ROWS):
                    pltpu.async_copy(tabs[f].at[idx_all.at[f, k, j]],
                                     bufs[b].at[pl.ds(j * LANES, LANES)],
                                     sems.at[ADD + b], add=True)

        def wait_adds(b):
            for f in range(1, NF):
                for j in range(CHUNK_ROWS):
                    pltpu.make_async_copy(
                        tabs[f].at[idx_all.at[f, 0, j]],
                        bufs[b].at[pl.ds(j * LANES, LANES)],
                        sems.at[ADD + b]).wait()

        def fire_out(b, k):
            pltpu.async_copy(bufs[b],
                             out.at[pl.ds((chunk0 + k) * CHUNK, CHUNK)],
                             sems.at[OUT + b])

        def wait_out(b):
            pltpu.make_async_copy(bufs[b], out.at[pl.ds(0, CHUNK)],
                                  sems.at[OUT + b]).wait()

        fire_f0(0, 0)

        def body(g, carry):
            c0 = 2 * g
            c1 = c0 + 1
            wait_f0(0)
            fire_adds(0, c0)

            @pl.when(g > 0)
            def _():
                wait_out(1)

            fire_f0(1, c1)
            wait_adds(0)
            fire_out(0, c0)
            wait_f0(1)
            fire_adds(1, c1)
            wait_out(0)
            fire_f0(0, c0 + 2)
            wait_adds(1)
            fire_out(1, c1)
            return carry

        lax.fori_loop(0, PAIRS, body, 0)

        wait_f0(0)
        fire_adds(0, CPW - 1)
        wait_out(1)
        wait_adds(0)
        fire_out(0, CPW - 1)
        wait_out(0)

    return gather_kernel(idx, P0, P1, P2, P3)


def kernel(tokens, E0, E1, E2, E3, Wp, bp):
    idx = tokens.astype(jnp.int32).reshape(BS, NF).T.reshape(
        NF, NCHUNKS, CHUNK_ROWS, LANES)
    P0, P1, P2, P3 = _tc_precompute(E0.T, E1.T, E2.T, E3.T, Wp,
                                    bp.reshape(1, EMBED_DIM))
    out = _sc_gather_sum(idx, P0, P1, P2, P3)
    return out.reshape(B, S, EMBED_DIM)


# speedup vs baseline: 10.2896x; 1.1661x over previous
import functools

import jax
import jax.numpy as jnp
from jax import lax
from jax.experimental import pallas as pl
from jax.experimental.pallas import tpu as pltpu
from jax.experimental.pallas import tpu_sc as plsc

CARD = 100000
B, S, NF = 1024, 200, 4
BS = B * S
DIMS = (32, 32, 16, 16)
TOTAL_DIM = 96
EMBED_DIM = 128

NW = 32
LANES = 128
CHUNK_ROWS = 2
CHUNK = CHUNK_ROWS * LANES
NCHUNKS = BS // CHUNK
CPW = NCHUNKS // NW
PAIRS = (CPW - 1) // 2


def _tc_precompute(E0t, E1t, E2t, E3t, Wp, bp2):
    R = 4096
    cdims = (((0,), (0,)), ((), ()))

    def mm(e0_ref, e1_ref, e2_ref, e3_ref, wp_ref, bp_ref,
           p0_ref, p1_ref, p2_ref, p3_ref):
        p0_ref[...] = lax.dot_general(
            e0_ref[...], wp_ref[0:32, :], cdims,
            preferred_element_type=jnp.float32) + bp_ref[...]
        p1_ref[...] = lax.dot_general(
            e1_ref[...], wp_ref[32:64, :], cdims,
            preferred_element_type=jnp.float32)
        p2_ref[...] = lax.dot_general(
            e2_ref[...], wp_ref[64:80, :], cdims,
            preferred_element_type=jnp.float32)
        p3_ref[...] = lax.dot_general(
            e3_ref[...], wp_ref[80:96, :], cdims,
            preferred_element_type=jnp.float32)

    pshape = jax.ShapeDtypeStruct((CARD, EMBED_DIM), jnp.float32)
    return pl.pallas_call(
        mm,
        grid=(pl.cdiv(CARD, R),),
        in_specs=[
            pl.BlockSpec((DIMS[0], R), lambda i: (0, i)),
            pl.BlockSpec((DIMS[1], R), lambda i: (0, i)),
            pl.BlockSpec((DIMS[2], R), lambda i: (0, i)),
            pl.BlockSpec((DIMS[3], R), lambda i: (0, i)),
            pl.BlockSpec((TOTAL_DIM, EMBED_DIM), lambda i: (0, 0)),
            pl.BlockSpec((1, EMBED_DIM), lambda i: (0, 0)),
        ],
        out_specs=[pl.BlockSpec((R, EMBED_DIM), lambda i: (i, 0))] * NF,
        out_shape=[pshape] * NF,
    )(E0t, E1t, E2t, E3t, Wp, bp2)


def _sc_gather_sum(idx, P0, P1, P2, P3):
    mesh = plsc.VectorSubcoreMesh(core_axis_name="c", subcore_axis_name="s")

    @functools.partial(
        pl.kernel,
        mesh=mesh,
        out_type=jax.ShapeDtypeStruct((BS, EMBED_DIM), jnp.float32),
        scratch_types=[
            pltpu.VMEM((NF, CPW, CHUNK_ROWS, LANES), jnp.int32),
            pltpu.VMEM((CHUNK, EMBED_DIM), jnp.float32),
            pltpu.VMEM((CHUNK, EMBED_DIM), jnp.float32),
            pltpu.SemaphoreType.DMA((6,)),
        ],
    )
    def gather_kernel(idx_hbm, t0, t1, t2, t3, out, idx_all, buf_a, buf_b,
                      sems):
        wid = lax.axis_index("c") * 16 + lax.axis_index("s")
        chunk0 = wid * CPW
        tabs = (t0, t1, t2, t3)
        bufs = (buf_a, buf_b)
        F0, ADD, OUT = 0, 2, 4

        for f in range(NF):
            pltpu.sync_copy(idx_hbm.at[f, pl.ds(chunk0, CPW)], idx_all.at[f])

        def fire_f0(b, k):
            for j in range(CHUNK_ROWS):
                pltpu.async_copy(tabs[0].at[idx_all.at[0, k, j]],
                                 bufs[b].at[pl.ds(j * LANES, LANES)],
                                 sems.at[F0 + b])

        def wait_f0(b):
            for j in range(CHUNK_ROWS):
                pltpu.make_async_copy(
                    tabs[0].at[idx_all.at[0, 0, j]],
                    bufs[b].at[pl.ds(j * LANES, LANES)],
                    sems.at[F0 + b]).wait()

        def fire_adds(b, k):
            for f in range(1, NF):
                for j in range(CHUNK_ROWS):
                    pltpu.async_copy(tabs[f].at[idx_all.at[f, k, j]],
                                     bufs[b].at[pl.ds(j * LANES, LANES)],
                                     sems.at[ADD + b], add=True)

        def wait_adds(b):
            for f in range(1, NF):
                for j in range(CHUNK_ROWS):
                    pltpu.make_async_copy(
                        tabs[f].at[idx_all.at[f, 0, j]],
                        bufs[b].at[pl.ds(j * LANES, LANES)],
                        sems.at[ADD + b]).wait()

        def fire_out(b, k):
            pltpu.async_copy(bufs[b],
                             out.at[pl.ds((chunk0 + k) * CHUNK, CHUNK)],
                             sems.at[OUT + b])

        def wait_out(b):
            pltpu.make_async_copy(bufs[b], out.at[pl.ds(0, CHUNK)],
                                  sems.at[OUT + b]).wait()

        fire_f0(0, 0)

        def body(g, carry):
            c0 = 2 * g
            c1 = c0 + 1
            wait_f0(0)
            fire_adds(0, c0)

            @pl.when(g > 0)
            def _():
                wait_out(1)

            fire_f0(1, c1)
            wait_adds(0)
            fire_out(0, c0)
            wait_f0(1)
            fire_adds(1, c1)
            wait_out(0)
            fire_f0(0, c0 + 2)
            wait_adds(1)
            fire_out(1, c1)
            return carry

        lax.fori_loop(0, PAIRS, body, 0)

        wait_f0(0)
        fire_adds(0, CPW - 1)
        wait_out(1)
        wait_adds(0)
        fire_out(0, CPW - 1)
        wait_out(0)

    return gather_kernel(idx, P0, P1, P2, P3)


def kernel(tokens, E0, E1, E2, E3, Wp, bp):
    idx = tokens.astype(jnp.int32).reshape(
        NCHUNKS, CHUNK_ROWS, LANES, NF).transpose(3, 0, 1, 2)
    P0, P1, P2, P3 = _tc_precompute(E0.T, E1.T, E2.T, E3.T, Wp,
                                    bp.reshape(1, EMBED_DIM))
    out = _sc_gather_sum(idx, P0, P1, P2, P3)
    return out.reshape(B, S, EMBED_DIM)
